# baseline (speedup 1.0000x reference)
import functools

import jax
import jax.numpy as jnp
from jax.experimental import pallas as pl


def _conv2d(x, w, b, stride, pad):
    y = jax.lax.conv_general_dilated(
        x, w, (stride, stride), [(pad, pad), (pad, pad)],
        dimension_numbers=('NCHW', 'OIHW', 'NCHW'))
    return y + b[None, :, None, None]


def _convT2d(x, w, b):
    cout = w.shape[1]
    wt = w.transpose(1, 0, 2, 3)
    banks = []
    for a in (0, 1):
        for bb in (0, 1):
            ti = (3, 1) if a == 0 else (2, 0)
            tj = (3, 1) if bb == 0 else (2, 0)
            banks.append(wt[:, :, ti, :][:, :, :, tj])
    kall = jnp.concatenate(banks, axis=0)
    y = jax.lax.conv_general_dilated(
        x, kall, (1, 1), [(1, 1), (1, 1)],
        dimension_numbers=('NCHW', 'OIHW', 'NCHW'))
    B, _, HP, WP = y.shape
    H, W = HP - 1, WP - 1
    o00 = y[:, 0 * cout:1 * cout, 0:H, 0:W]
    o01 = y[:, 1 * cout:2 * cout, 0:H, 1:W + 1]
    o10 = y[:, 2 * cout:3 * cout, 1:H + 1, 0:W]
    o11 = y[:, 3 * cout:4 * cout, 1:H + 1, 1:W + 1]
    row0 = jnp.stack([o00, o01], axis=-1)
    row1 = jnp.stack([o10, o11], axis=-1)
    out = jnp.stack([row0, row1], axis=3).reshape(B, cout, 2 * H, 2 * W)
    return out + b[None, :, None, None]


def _rvq_body(z_ref, cb1_ref, cbcat_ref, cbn_ref,
              qout_ref, idx_ref, loss_ref, *, nq, k):
    t = z_ref.shape[0]
    z = z_ref[...]

    @pl.when(pl.program_id(0) == 0)
    def _init():
        loss_ref[...] = jnp.zeros_like(loss_ref)

    residual = z
    qsum = jnp.zeros_like(z)
    lane = jax.lax.broadcasted_iota(jnp.int32, (t, k), 1)
    lane3 = jnp.bitwise_and(
        jax.lax.broadcasted_iota(jnp.int32, (t, 3 * k), 1), k - 1)
    loss_lane = jax.lax.broadcasted_iota(jnp.int32, loss_ref.shape, 1)
    idx_rows = []
    loss_acc = jnp.zeros(loss_ref.shape, jnp.float32)
    for q in range(nq):
        cb1 = cb1_ref[q]
        dot = jnp.dot(residual.astype(jnp.bfloat16), cb1.T,
                      preferred_element_type=jnp.float32)
        d = (jnp.sum(residual * residual, axis=1, keepdims=True)
             - 2.0 * dot) + cbn_ref[q][None, :]
        dmin = jnp.min(d, axis=1, keepdims=True)
        idx = jnp.min(jnp.where(d == dmin, lane, k), axis=1)
        onehot3 = (lane3 == idx[:, None]).astype(jnp.float32)
        quant = jnp.dot(onehot3, cbcat_ref[q],
                        preferred_element_type=jnp.float32)
        loss_q = jnp.sum((quant - residual) ** 2)
        loss_acc = loss_acc + jnp.where(loss_lane == q, loss_q, 0.0)
        idx_rows.append(idx.astype(jnp.int32))
        qsum = qsum + quant
        residual = residual - quant
    qout_ref[...] = qsum
    idx_ref[...] = jnp.stack(idx_rows, axis=0)
    loss_ref[...] += loss_acc


def _residual_vq_pallas(z_flat, codebooks):
    b, n, d = z_flat.shape
    nq, k, _ = codebooks.shape
    ntot = b * n
    tile = 256
    assert ntot % tile == 0
    grid = ntot // tile
    z2 = z_flat.reshape(ntot, d)
    cbn = jnp.sum(codebooks * codebooks, axis=-1)
    def _trunc8(v):
        bits = jax.lax.bitcast_convert_type(v, jnp.int32)
        return jax.lax.bitcast_convert_type(
            jnp.bitwise_and(bits, jnp.int32(-65536)), jnp.float32)

    p1 = _trunc8(codebooks)
    r1 = codebooks - p1
    p2 = _trunc8(r1)
    p3 = r1 - p2
    cbcat = jnp.concatenate([p1, p2, p3], axis=1)
    cb1 = codebooks.astype(jnp.bfloat16)

    qout, idxt, loss_part = pl.pallas_call(
        functools.partial(_rvq_body, nq=nq, k=k),
        grid=(grid,),
        in_specs=[
            pl.BlockSpec((tile, d), lambda i: (i, 0)),
            pl.BlockSpec((nq, k, d), lambda i: (0, 0, 0)),
            pl.BlockSpec((nq, 3 * k, d), lambda i: (0, 0, 0)),
            pl.BlockSpec((nq, k), lambda i: (0, 0)),
        ],
        out_specs=[
            pl.BlockSpec((tile, d), lambda i: (i, 0)),
            pl.BlockSpec((nq, tile), lambda i: (0, i)),
            pl.BlockSpec((8, 128), lambda i: (0, 0)),
        ],
        out_shape=[
            jax.ShapeDtypeStruct((ntot, d), jnp.float32),
            jax.ShapeDtypeStruct((nq, ntot), jnp.int32),
            jax.ShapeDtypeStruct((8, 128), jnp.float32),
        ],
    )(z2, cb1, cbcat, cbn)

    quant_out = qout.reshape(b, n, d)
    indices = idxt.reshape(nq, b, n).transpose(1, 2, 0)
    losses = loss_part[0, :nq] / jnp.float32(b * n * d)
    return quant_out, indices, losses


def kernel(x, ew1, eb1, ew2, eb2, ew3, eb3, ew4, eb4,
           dw0, db0, dw1, db1, dw2, db2, dw3, db3, codebooks):
    z = jax.nn.relu(_conv2d(x, ew1, eb1, 2, 1))
    z = jax.nn.relu(_conv2d(z, ew2, eb2, 2, 1))
    z = jax.nn.relu(_conv2d(z, ew3, eb3, 1, 1))
    z = _conv2d(z, ew4, eb4, 1, 1)
    B, C, H, W = z.shape
    z_p = z.transpose(0, 2, 3, 1)
    z_flat = z_p.reshape(B, H * W, C)
    quant_flat, indices, commit_loss = _residual_vq_pallas(z_flat, codebooks)
    quantized = quant_flat.reshape(B, H, W, C).transpose(0, 3, 1, 2)
    r = jax.nn.relu(_conv2d(quantized, dw0, db0, 1, 1))
    r = jax.nn.relu(_convT2d(r, dw1, db1))
    r = jax.nn.relu(_convT2d(r, dw2, db2))
    recon = jnp.tanh(_convT2d(r, dw3, db3))
    nq = codebooks.shape[0]
    indices_out = indices.transpose(0, 2, 1).reshape(B, nq, H, W)
    return recon, indices_out, commit_loss, quantized


# speedup vs baseline: 2.1558x; 2.1558x over previous
import functools

import jax
import jax.numpy as jnp
from jax.experimental import pallas as pl


def _conv2d(x, w, b, stride, pad):
    y = jax.lax.conv_general_dilated(
        x, w, (stride, stride), [(pad, pad), (pad, pad)],
        dimension_numbers=('NCHW', 'OIHW', 'NCHW'))
    return y + b[None, :, None, None]


def _convT2d(x, w, b, stride, pad):
    kh, kw = w.shape[2], w.shape[3]
    w2 = jnp.flip(w, (2, 3)).transpose(1, 0, 2, 3)
    y = jax.lax.conv_general_dilated(
        x, w2, (1, 1),
        [(kh - 1 - pad, kh - 1 - pad), (kw - 1 - pad, kw - 1 - pad)],
        lhs_dilation=(stride, stride),
        dimension_numbers=('NCHW', 'OIHW', 'NCHW'))
    return y + b[None, :, None, None]


def _rvq_body(z_ref, cb1_ref, cbcat_ref, cbn_ref,
              qout_ref, idx_ref, loss_ref, *, nq, k):
    t = z_ref.shape[0]
    z = z_ref[...]

    @pl.when(pl.program_id(0) == 0)
    def _init():
        loss_ref[...] = jnp.zeros_like(loss_ref)

    residual = z
    qsum = jnp.zeros_like(z)
    lane = jax.lax.broadcasted_iota(jnp.int32, (t, k), 1)
    lane3 = jnp.bitwise_and(
        jax.lax.broadcasted_iota(jnp.int32, (t, 3 * k), 1), k - 1)
    loss_lane = jax.lax.broadcasted_iota(jnp.int32, loss_ref.shape, 1)
    idx_rows = []
    loss_acc = jnp.zeros(loss_ref.shape, jnp.float32)
    for q in range(nq):
        cb1 = cb1_ref[q]
        dot = jnp.dot(residual.astype(jnp.bfloat16), cb1.T,
                      preferred_element_type=jnp.float32)
        d = (jnp.sum(residual * residual, axis=1, keepdims=True)
             - 2.0 * dot) + cbn_ref[q][None, :]
        dmin = jnp.min(d, axis=1, keepdims=True)
        idx = jnp.min(jnp.where(d == dmin, lane, k), axis=1)
        onehot3 = (lane3 == idx[:, None]).astype(jnp.float32)
        quant = jnp.dot(onehot3, cbcat_ref[q],
                        preferred_element_type=jnp.float32)
        loss_q = jnp.sum((quant - residual) ** 2)
        loss_acc = loss_acc + jnp.where(loss_lane == q, loss_q, 0.0)
        idx_rows.append(idx.astype(jnp.int32))
        qsum = qsum + quant
        residual = residual - quant
    qout_ref[...] = qsum
    idx_ref[...] = jnp.stack(idx_rows, axis=0)
    loss_ref[...] += loss_acc


def _residual_vq_pallas(z_flat, codebooks):
    b, n, d = z_flat.shape
    nq, k, _ = codebooks.shape
    ntot = b * n
    tile = 896
    assert ntot % tile == 0
    grid = ntot // tile
    z2 = z_flat.reshape(ntot, d)
    cbn = jnp.sum(codebooks * codebooks, axis=-1)
    def _trunc8(v):
        bits = jax.lax.bitcast_convert_type(v, jnp.int32)
        return jax.lax.bitcast_convert_type(
            jnp.bitwise_and(bits, jnp.int32(-65536)), jnp.float32)

    p1 = _trunc8(codebooks)
    r1 = codebooks - p1
    p2 = _trunc8(r1)
    p3 = r1 - p2
    cbcat = jnp.concatenate([p1, p2, p3], axis=1)
    cb1 = codebooks.astype(jnp.bfloat16)

    qout, idxt, loss_part = pl.pallas_call(
        functools.partial(_rvq_body, nq=nq, k=k),
        grid=(grid,),
        in_specs=[
            pl.BlockSpec((tile, d), lambda i: (i, 0)),
            pl.BlockSpec((nq, k, d), lambda i: (0, 0, 0)),
            pl.BlockSpec((nq, 3 * k, d), lambda i: (0, 0, 0)),
            pl.BlockSpec((nq, k), lambda i: (0, 0)),
        ],
        out_specs=[
            pl.BlockSpec((tile, d), lambda i: (i, 0)),
            pl.BlockSpec((nq, tile), lambda i: (0, i)),
            pl.BlockSpec((8, 128), lambda i: (0, 0)),
        ],
        out_shape=[
            jax.ShapeDtypeStruct((ntot, d), jnp.float32),
            jax.ShapeDtypeStruct((nq, ntot), jnp.int32),
            jax.ShapeDtypeStruct((8, 128), jnp.float32),
        ],
    )(z2, cb1, cbcat, cbn)

    quant_out = qout.reshape(b, n, d)
    indices = idxt.reshape(nq, b, n).transpose(1, 2, 0)
    losses = loss_part[0, :nq] / jnp.float32(b * n * d)
    return quant_out, indices, losses


def kernel(x, ew1, eb1, ew2, eb2, ew3, eb3, ew4, eb4,
           dw0, db0, dw1, db1, dw2, db2, dw3, db3, codebooks):
    z = jax.nn.relu(_conv2d(x, ew1, eb1, 2, 1))
    z = jax.nn.relu(_conv2d(z, ew2, eb2, 2, 1))
    z = jax.nn.relu(_conv2d(z, ew3, eb3, 1, 1))
    z = _conv2d(z, ew4, eb4, 1, 1)
    B, C, H, W = z.shape
    z_p = z.transpose(0, 2, 3, 1)
    z_flat = z_p.reshape(B, H * W, C)
    quant_flat, indices, commit_loss = _residual_vq_pallas(z_flat, codebooks)
    quantized = quant_flat.reshape(B, H, W, C).transpose(0, 3, 1, 2)
    r = jax.nn.relu(_conv2d(quantized, dw0, db0, 1, 1))
    r = jax.nn.relu(_convT2d(r, dw1, db1, 2, 1))
    r = jax.nn.relu(_convT2d(r, dw2, db2, 2, 1))
    recon = jnp.tanh(_convT2d(r, dw3, db3, 2, 1))
    nq = codebooks.shape[0]
    indices_out = indices.transpose(0, 2, 1).reshape(B, nq, H, W)
    return recon, indices_out, commit_loss, quantized
